# Initial kernel scaffold; baseline (speedup 1.0000x reference)
import jax
import jax.numpy as jnp
from jax.experimental import pallas as pl


def kernel(token_ids, weight):
    raise NotImplementedError("write your pallas kernel here")


# speedup vs baseline: 1.8750x; 1.8750x over previous
import functools

import jax
import jax.numpy as jnp
from jax import lax
from jax.experimental import pallas as pl
from jax.experimental.pallas import tpu as pltpu
from jax.experimental.pallas import tpu_sc as plsc

NUM_CORES = 2
NUM_SUBCORES = 16
NW = NUM_CORES * NUM_SUBCORES
CHUNK = 512


def _build(B, V, D):
    assert B % NW == 0
    pw = B // NW
    assert pw % (2 * CHUNK) == 0
    nchunks = pw // CHUNK
    npairs = nchunks // 2

    mesh = plsc.VectorSubcoreMesh(
        core_axis_name="c", subcore_axis_name="s",
        num_cores=NUM_CORES, num_subcores=NUM_SUBCORES)

    @functools.partial(
        pl.kernel,
        out_type=jax.ShapeDtypeStruct((B, D), jnp.float32),
        mesh=mesh,
        scratch_types=[
            pltpu.VMEM((pw,), jnp.int32),
            pltpu.VMEM((2, CHUNK, D), jnp.float32),
            pltpu.SemaphoreType.DMA,
            pltpu.SemaphoreType.DMA,
        ],
        compiler_params=pltpu.CompilerParams(use_tc_tiling_on_sc=False),
    )
    def emb(weight_hbm, idx_hbm, out_hbm, idx_v, rows_v, sem0, sem1):
        wid = lax.axis_index("s") * NUM_CORES + lax.axis_index("c")
        base = wid * pw
        pltpu.sync_copy(idx_hbm.at[pl.ds(base, pw)], idx_v)
        sems = (sem0, sem1)

        def gather(g, b):
            pltpu.async_copy(
                weight_hbm.at[idx_v.at[pl.ds(g * CHUNK, CHUNK)]],
                rows_v.at[b], sems[b])

        def wait_gather(g, b):
            pltpu.make_async_copy(
                weight_hbm.at[idx_v.at[pl.ds(g * CHUNK, CHUNK)]],
                rows_v.at[b], sems[b]).wait()

        gather(0, 0)
        gather(1, 1)

        def step(gp, _):
            for b in range(2):
                g = gp * 2 + b
                wait_gather(g, b)
                pltpu.sync_copy(rows_v.at[b],
                                out_hbm.at[pl.ds(base + g * CHUNK, CHUNK)])
                gather(g + 2, b)
            return _

        lax.fori_loop(0, npairs - 1, step, 0, unroll=False)

        for b in range(2):
            g = (npairs - 1) * 2 + b
            wait_gather(g, b)
            pltpu.sync_copy(rows_v.at[b],
                            out_hbm.at[pl.ds(base + g * CHUNK, CHUNK)])

    return emb


def kernel(token_ids, weight):
    B = token_ids.size
    V, D = weight.shape
    idx = jnp.reshape(token_ids, (B,)).astype(jnp.int32)
    out = _build(B, V, D)(weight, idx)
    return jnp.reshape(out, token_ids.shape + (D,))
